# baseline (speedup 1.0000x reference)
import functools

import jax
import jax.numpy as jnp
from jax import lax
from jax.experimental import pallas as pl
from jax.experimental.pallas import tpu as pltpu
from jax.experimental.pallas import tpu_sc as plsc

NC = 2
NS = 16
NW = NC * NS

B, T = 4096, 200
D = 64
V = 1000000
G = 128
NBB = B // G
TB = 8
NSUP = (T // TB) * NBB
SUP_W = NSUP // NW
STEPS = SUP_W * TB


@functools.partial(
    pl.kernel,
    out_type=jax.ShapeDtypeStruct((T, D, B), jnp.float32),
    mesh=plsc.VectorSubcoreMesh(core_axis_name="c", subcore_axis_name="s"),
    scratch_types=[
        pltpu.VMEM((STEPS, G), jnp.int32),
        pltpu.VMEM((3, G, 128), jnp.float32),
        pltpu.VMEM((2, D, G), jnp.float32),
        pltpu.SemaphoreType.DMA,
        pltpu.SemaphoreType.DMA,
    ],
    compiler_params=pltpu.CompilerParams(
        use_tc_tiling_on_sc=True, needs_layout_passes=False
    ),
)
def _gather_kernel(ints_t, tw, out_hbm, ibuf, gbuf, tr, gsem, wsem):
    w = lax.axis_index("s") * NC + lax.axis_index("c")
    lanes = lax.iota(jnp.int32, 16)

    for m in range(SUP_W):
        sup = w * SUP_W + m
        tb8 = (sup // NBB) * TB
        bb = sup % NBB
        pltpu.sync_copy(
            ints_t.at[pl.ds(tb8, TB), pl.ds(bb * G, G)],
            ibuf.at[pl.ds(m * TB, TB), :],
        )

    def issue_gather(s):
        pltpu.async_copy(tw.at[ibuf.at[s]], gbuf.at[lax.rem(s, 3)], gsem)

    issue_gather(0)
    issue_gather(1)

    def body(s, carry):
        bt = lax.rem(s, 2)
        sup = w * SUP_W + s // TB
        t = (sup // NBB) * TB + lax.rem(s, TB)
        bb = lax.rem(sup, NBB)

        @pl.when(s >= 2)
        def _():
            pltpu.make_async_copy(
                tr.at[bt], out_hbm.at[0, :, pl.ds(0, G)], wsem
            ).wait()

        @pl.when(s + 2 < STEPS)
        def _():
            issue_gather(s + 2)

        pltpu.make_async_copy(
            tw.at[ibuf.at[s]], gbuf.at[lax.rem(s, 3)], gsem
        ).wait()

        src = gbuf.at[lax.rem(s, 3)]
        dst = tr.at[bt]
        for ig in range(G // 16):
            rows = lanes + (ig * 16)
            for d0 in range(0, D, 8):
                vals = [
                    plsc.load_gather(
                        src, [rows, jnp.full((16,), d0 + k, jnp.int32)]
                    )
                    for k in range(8)
                ]
                for k in range(8):
                    dst[d0 + k, pl.ds(ig * 16, 16)] = vals[k]

        pltpu.async_copy(dst, out_hbm.at[t, :, pl.ds(bb * G, G)], wsem)
        return carry

    lax.fori_loop(0, STEPS, body, 0)

    pltpu.make_async_copy(tr.at[0], out_hbm.at[0, :, pl.ds(0, G)], wsem).wait()
    pltpu.make_async_copy(tr.at[0], out_hbm.at[0, :, pl.ds(0, G)], wsem).wait()


def kernel(integers, table):
    ints_t = integers.T
    tw = jnp.pad(table, ((0, 0), (0, 64)))
    out = _gather_kernel(ints_t, tw)
    return out.transpose(2, 0, 1)


# speedup vs baseline: 1.6415x; 1.6415x over previous
import functools

import jax
import jax.numpy as jnp
from jax import lax
from jax.experimental import pallas as pl
from jax.experimental.pallas import tpu as pltpu
from jax.experimental.pallas import tpu_sc as plsc

NC = 2
NS = 16
NW = NC * NS

B, T = 4096, 200
D = 64
V = 1000000
G = 128
NBB = B // G
TB = 8
NSUP = (T // TB) * NBB
SUP_W = NSUP // NW
STEPS = SUP_W * TB


@functools.partial(
    pl.kernel,
    out_type=jax.ShapeDtypeStruct((T, D, B), jnp.float32),
    mesh=plsc.VectorSubcoreMesh(core_axis_name="c", subcore_axis_name="s"),
    scratch_types=[
        pltpu.VMEM((STEPS, G), jnp.int32),
        pltpu.VMEM((3, G, 128), jnp.float32),
        pltpu.VMEM((2, D, G), jnp.float32),
        pltpu.SemaphoreType.DMA,
        pltpu.SemaphoreType.DMA,
    ],
    compiler_params=pltpu.CompilerParams(
        use_tc_tiling_on_sc=True, needs_layout_passes=False
    ),
)
def _gather_kernel(ints_t, tw, out_hbm, ibuf, gbuf, tr, gsem, wsem):
    w = lax.axis_index("s") * NC + lax.axis_index("c")
    lanes = lax.iota(jnp.int32, 16)

    for m in range(SUP_W):
        sup = w * SUP_W + m
        tb8 = (sup // NBB) * TB
        bb = sup % NBB
        pltpu.sync_copy(
            ints_t.at[pl.ds(tb8, TB), pl.ds(bb * G, G)],
            ibuf.at[pl.ds(m * TB, TB), :],
        )

    def issue_gather(s):
        pltpu.async_copy(tw.at[ibuf.at[s]], gbuf.at[lax.rem(s, 3)], gsem)

    issue_gather(0)
    issue_gather(1)

    def body(s, carry):
        bt = lax.rem(s, 2)
        sup = w * SUP_W + s // TB
        t = (sup // NBB) * TB + lax.rem(s, TB)
        bb = lax.rem(sup, NBB)

        @pl.when(s >= 2)
        def _():
            pltpu.make_async_copy(
                tr.at[bt], out_hbm.at[0, :, pl.ds(0, G)], wsem
            ).wait()

        @pl.when(s + 2 < STEPS)
        def _():
            issue_gather(s + 2)

        pltpu.make_async_copy(
            tw.at[ibuf.at[s]], gbuf.at[lax.rem(s, 3)], gsem
        ).wait()

        src = gbuf.at[lax.rem(s, 3)]
        dst = tr.at[bt]
        diags = [jnp.bitwise_and(lanes + k, 15) for k in range(16)]

        @pl.loop(0, G, step=16)
        def _(i0):
            for d0 in range(0, D, 16):
                cv = lanes + d0
                for k0 in range(0, 16, 8):
                    rvs = [diags[k0 + k] + i0 for k in range(8)]
                    vals = [
                        plsc.load_gather(src, [rvs[k], cv]) for k in range(8)
                    ]
                    for k in range(8):
                        plsc.store_scatter(dst, [cv, rvs[k]], vals[k])

        pltpu.async_copy(dst, out_hbm.at[t, :, pl.ds(bb * G, G)], wsem)
        return carry

    lax.fori_loop(0, STEPS, body, 0)

    pltpu.make_async_copy(tr.at[0], out_hbm.at[0, :, pl.ds(0, G)], wsem).wait()
    pltpu.make_async_copy(tr.at[0], out_hbm.at[0, :, pl.ds(0, G)], wsem).wait()


def kernel(integers, table):
    ints_t = integers.T
    tw = jnp.pad(table, ((0, 0), (0, 64)))
    out = _gather_kernel(ints_t, tw)
    return out.transpose(2, 0, 1)


# speedup vs baseline: 1.7216x; 1.0488x over previous
import functools

import jax
import jax.numpy as jnp
from jax import lax
from jax.experimental import pallas as pl
from jax.experimental.pallas import tpu as pltpu
from jax.experimental.pallas import tpu_sc as plsc

NC = 2
NS = 16
NW = NC * NS

B, T = 4096, 200
D = 64
V = 1000000
G = 128
NBB = B // G
TB = 8
NSUP = (T // TB) * NBB
SUP_W = NSUP // NW
STEPS = SUP_W * TB


@functools.partial(
    pl.kernel,
    out_type=jax.ShapeDtypeStruct((T, D, B), jnp.float32),
    mesh=plsc.VectorSubcoreMesh(core_axis_name="c", subcore_axis_name="s"),
    scratch_types=[
        pltpu.VMEM((STEPS, G), jnp.int32),
        pltpu.VMEM((3, G, 128), jnp.float32),
        pltpu.VMEM((2, D, G), jnp.float32),
        pltpu.SemaphoreType.DMA,
        pltpu.SemaphoreType.DMA,
    ],
    compiler_params=pltpu.CompilerParams(
        use_tc_tiling_on_sc=True, needs_layout_passes=False
    ),
)
def _gather_kernel(ints_t, tw, out_hbm, ibuf, gbuf, tr, gsem, wsem):
    w = lax.axis_index("s") * NC + lax.axis_index("c")
    lanes = lax.iota(jnp.int32, 16)

    for m in range(SUP_W):
        sup = w * SUP_W + m
        tb8 = (sup // NBB) * TB
        bb = sup % NBB
        pltpu.sync_copy(
            ints_t.at[pl.ds(tb8, TB), pl.ds(bb * G, G)],
            ibuf.at[pl.ds(m * TB, TB), :],
        )

    def issue_gather(s):
        pltpu.async_copy(tw.at[ibuf.at[s]], gbuf.at[lax.rem(s, 3)], gsem)

    issue_gather(0)
    issue_gather(1)

    def body(s, carry):
        bt = lax.rem(s, 2)
        sup = w * SUP_W + s // TB
        t = (sup // NBB) * TB + lax.rem(s, TB)
        bb = lax.rem(sup, NBB)

        @pl.when(s >= 2)
        def _():
            pltpu.make_async_copy(
                tr.at[bt], out_hbm.at[0, :, pl.ds(0, G)], wsem
            ).wait()

        @pl.when(s + 2 < STEPS)
        def _():
            issue_gather(s + 2)

        pltpu.make_async_copy(
            tw.at[ibuf.at[s]], gbuf.at[lax.rem(s, 3)], gsem
        ).wait()

        src = gbuf.at[lax.rem(s, 3)]
        dst = tr.at[bt]
        diags = [jnp.bitwise_and(lanes + k, 15) for k in range(16)]

        @pl.loop(0, G, step=16)
        def _(i0):
            for d0 in range(0, D, 16):
                cv = lanes + d0
                for k0 in range(0, 16, 8):
                    rvs = [diags[k0 + k] + i0 for k in range(8)]
                    vals = [
                        plsc.load_gather(src, [rvs[k], cv]) for k in range(8)
                    ]
                    for k in range(8):
                        plsc.store_scatter(dst, [cv, rvs[k]], vals[k])

        pltpu.async_copy(dst, out_hbm.at[t, :, pl.ds(bb * G, G)], wsem)
        return carry

    lax.fori_loop(0, STEPS, body, 0)

    pltpu.make_async_copy(tr.at[0], out_hbm.at[0, :, pl.ds(0, G)], wsem).wait()
    pltpu.make_async_copy(tr.at[0], out_hbm.at[0, :, pl.ds(0, G)], wsem).wait()


NBLK = (V + 127) // 128


@functools.partial(
    pl.kernel,
    out_type=jax.ShapeDtypeStruct((NBLK * 128, 128), jnp.float32),
    mesh=plsc.VectorSubcoreMesh(core_axis_name="c", subcore_axis_name="s"),
    scratch_types=[
        pltpu.VMEM((2, D, 128), jnp.float32),
        pltpu.VMEM((2, 128, 128), jnp.float32),
        pltpu.SemaphoreType.DMA,
        pltpu.SemaphoreType.DMA,
    ],
    compiler_params=pltpu.CompilerParams(
        use_tc_tiling_on_sc=True,
        needs_layout_passes=False,
        disable_bounds_checks=True,
    ),
)
def _pack_kernel(tt, tp, vin, vout, rsem, wsem):
    w = lax.axis_index("s") * NC + lax.axis_index("c")
    lanes = lax.iota(jnp.int32, 16)
    base, extra = divmod(NBLK, NW)
    cnt = base + jnp.where(w < extra, 1, 0)
    first = base * w + jnp.minimum(w, extra)

    def col0(i):
        return pl.multiple_of((first + i) * 128, 128)

    def issue_read(i):
        pltpu.async_copy(
            tt.at[:, pl.ds(col0(i), 128)], vin.at[lax.rem(i, 2)], rsem
        )

    issue_read(0)

    def body(i, carry):
        b = lax.rem(i, 2)

        @pl.when(i + 1 < cnt)
        def _():
            issue_read(i + 1)

        pltpu.make_async_copy(
            tt.at[:, pl.ds(col0(i), 128)], vin.at[b], rsem
        ).wait()

        @pl.when(i >= 2)
        def _():
            pltpu.make_async_copy(vout.at[b], tp.at[pl.ds(0, 128), :], wsem).wait()

        src = vin.at[b]
        dstp = vout.at[b]
        diags = [jnp.bitwise_and(lanes + k, 15) for k in range(16)]

        @pl.loop(0, 128, step=16)
        def _(x0):
            rvl = lanes + jnp.bitwise_and(x0, 63)
            rvs = lanes + x0
            for j0 in range(0, 128, 16):
                for k0 in range(0, 16, 8):
                    cvs = [diags[k0 + k] + j0 for k in range(8)]
                    vals = [
                        plsc.load_gather(src, [rvl, cvs[k]]) for k in range(8)
                    ]
                    for k in range(8):
                        plsc.store_scatter(dstp, [cvs[k], rvs], vals[k])

        pltpu.async_copy(dstp, tp.at[pl.ds(col0(i), 128), :], wsem)
        return carry

    lax.fori_loop(0, cnt, body, 0)

    pltpu.make_async_copy(vout.at[0], tp.at[pl.ds(0, 128), :], wsem).wait()
    pltpu.make_async_copy(vout.at[0], tp.at[pl.ds(0, 128), :], wsem).wait()


def kernel(integers, table):
    ints_t = integers.T
    tt = jnp.swapaxes(table, 0, 1)
    tp = _pack_kernel(tt)
    out = _gather_kernel(ints_t, tp)
    return out.transpose(2, 0, 1)


# speedup vs baseline: 2.2744x; 1.3211x over previous
import functools

import jax
import jax.numpy as jnp
from jax import lax
from jax.experimental import pallas as pl
from jax.experimental.pallas import tpu as pltpu
from jax.experimental.pallas import tpu_sc as plsc

NC = 2
NS = 16
NW = NC * NS

B, T = 4096, 200
D = 64
V = 1000000
G = 128
NBB = B // G
TB = 8
NSUP = (T // TB) * NBB
SUP_W = NSUP // NW
STEPS = SUP_W * TB


@functools.partial(
    pl.kernel,
    out_type=jax.ShapeDtypeStruct((T, D, B), jnp.float32),
    mesh=plsc.VectorSubcoreMesh(core_axis_name="c", subcore_axis_name="s"),
    scratch_types=[
        pltpu.VMEM((STEPS, G), jnp.int32),
        pltpu.VMEM((3, G, 128), jnp.float32),
        pltpu.VMEM((2, D, G), jnp.float32),
        pltpu.SemaphoreType.DMA,
        pltpu.SemaphoreType.DMA,
    ],
    compiler_params=pltpu.CompilerParams(
        use_tc_tiling_on_sc=True, needs_layout_passes=False
    ),
)
def _gather_kernel(ints_t, tw, out_hbm, ibuf, gbuf, tr, gsem, wsem):
    w = lax.axis_index("s") * NC + lax.axis_index("c")
    lanes = lax.iota(jnp.int32, 16)

    for m in range(SUP_W):
        sup = w * SUP_W + m
        tb8 = (sup // NBB) * TB
        bb = sup % NBB
        pltpu.sync_copy(
            ints_t.at[pl.ds(tb8, TB), pl.ds(bb * G, G)],
            ibuf.at[pl.ds(m * TB, TB), :],
        )

    def issue_gather(s):
        pltpu.async_copy(tw.at[ibuf.at[s]], gbuf.at[lax.rem(s, 3)], gsem)

    issue_gather(0)
    issue_gather(1)

    def body(s, carry):
        bt = lax.rem(s, 2)
        sup = w * SUP_W + s // TB
        t = (sup // NBB) * TB + lax.rem(s, TB)
        bb = lax.rem(sup, NBB)

        @pl.when(s >= 2)
        def _():
            pltpu.make_async_copy(
                tr.at[bt], out_hbm.at[0, :, pl.ds(0, G)], wsem
            ).wait()

        @pl.when(s + 2 < STEPS)
        def _():
            issue_gather(s + 2)

        pltpu.make_async_copy(
            tw.at[ibuf.at[s]], gbuf.at[lax.rem(s, 3)], gsem
        ).wait()

        src = gbuf.at[lax.rem(s, 3)]
        dst = tr.at[bt]
        diags = [jnp.bitwise_and(lanes + k, 15) for k in range(16)]

        @pl.loop(0, G, step=16)
        def _(i0):
            for d0 in range(0, D, 16):
                cv = lanes + d0
                for k0 in range(0, 16, 8):
                    rvs = [diags[k0 + k] + i0 for k in range(8)]
                    vals = [
                        plsc.load_gather(src, [rvs[k], cv]) for k in range(8)
                    ]
                    for k in range(8):
                        plsc.store_scatter(dst, [cv, rvs[k]], vals[k])

        pltpu.async_copy(dst, out_hbm.at[t, :, pl.ds(bb * G, G)], wsem)
        return carry

    lax.fori_loop(0, STEPS, body, 0)

    pltpu.make_async_copy(tr.at[0], out_hbm.at[0, :, pl.ds(0, G)], wsem).wait()
    pltpu.make_async_copy(tr.at[0], out_hbm.at[0, :, pl.ds(0, G)], wsem).wait()


NBLK = (V + 127) // 128


@functools.partial(
    pl.kernel,
    out_type=jax.ShapeDtypeStruct((NBLK * 128, 128), jnp.float32),
    mesh=plsc.VectorSubcoreMesh(core_axis_name="c", subcore_axis_name="s"),
    scratch_types=[
        pltpu.VMEM((2, D, 128), jnp.float32),
        pltpu.VMEM((2, 128, 128), jnp.float32),
        pltpu.SemaphoreType.DMA,
        pltpu.SemaphoreType.DMA,
    ],
    compiler_params=pltpu.CompilerParams(
        use_tc_tiling_on_sc=True,
        needs_layout_passes=False,
        disable_bounds_checks=True,
    ),
)
def _pack_kernel(tt, tp, vin, vout, rsem, wsem):
    w = lax.axis_index("s") * NC + lax.axis_index("c")
    lanes = lax.iota(jnp.int32, 16)
    base, extra = divmod(NBLK, NW)
    cnt = base + jnp.where(w < extra, 1, 0)
    first = base * w + jnp.minimum(w, extra)

    def col0(i):
        return pl.multiple_of((first + i) * 128, 128)

    def issue_read(i):
        pltpu.async_copy(
            tt.at[:, pl.ds(col0(i), 128)], vin.at[lax.rem(i, 2)], rsem
        )

    issue_read(0)

    def body(i, carry):
        b = lax.rem(i, 2)

        @pl.when(i + 1 < cnt)
        def _():
            issue_read(i + 1)

        pltpu.make_async_copy(
            tt.at[:, pl.ds(col0(i), 128)], vin.at[b], rsem
        ).wait()

        @pl.when(i >= 2)
        def _():
            pltpu.make_async_copy(vout.at[b], tp.at[pl.ds(0, 128), :], wsem).wait()

        src = vin.at[b]
        dstp = vout.at[b]
        diags = [jnp.bitwise_and(lanes + k, 15) for k in range(16)]

        @pl.loop(0, D, step=16)
        def _(x0):
            rvl = lanes + x0
            rvs = lanes + x0
            for j0 in range(0, 128, 16):
                for k0 in range(0, 16, 8):
                    cvs = [diags[k0 + k] + j0 for k in range(8)]
                    vals = [
                        plsc.load_gather(src, [rvl, cvs[k]]) for k in range(8)
                    ]
                    for k in range(8):
                        plsc.store_scatter(dstp, [cvs[k], rvs], vals[k])

        pltpu.async_copy(dstp, tp.at[pl.ds(col0(i), 128), :], wsem)
        return carry

    lax.fori_loop(0, cnt, body, 0)

    pltpu.make_async_copy(vout.at[0], tp.at[pl.ds(0, 128), :], wsem).wait()
    pltpu.make_async_copy(vout.at[0], tp.at[pl.ds(0, 128), :], wsem).wait()


def kernel(integers, table):
    ints_t = integers.T
    tt = jnp.swapaxes(table, 0, 1)
    tp = _pack_kernel(tt)
    out = _gather_kernel(ints_t, tp)
    return out.transpose(2, 0, 1)


# speedup vs baseline: 2.3346x; 1.0265x over previous
import functools

import jax
import jax.numpy as jnp
from jax import lax
from jax.experimental import pallas as pl
from jax.experimental.pallas import tpu as pltpu
from jax.experimental.pallas import tpu_sc as plsc

NC = 2
NS = 16
NW = NC * NS

B, T = 4096, 200
D = 64
V = 1000000
G = 128
NBB = B // G
TB = 8
NSUP = (T // TB) * NBB
SUP_W = NSUP // NW
STEPS = SUP_W * TB


@functools.partial(
    pl.kernel,
    out_type=jax.ShapeDtypeStruct((T, D, B), jnp.float32),
    mesh=plsc.VectorSubcoreMesh(core_axis_name="c", subcore_axis_name="s"),
    scratch_types=[
        pltpu.VMEM((STEPS, G), jnp.int32),
        pltpu.VMEM((3, G, 128), jnp.float32),
        pltpu.VMEM((2, D, G), jnp.float32),
        pltpu.SemaphoreType.DMA,
        pltpu.SemaphoreType.DMA,
    ],
    compiler_params=pltpu.CompilerParams(
        use_tc_tiling_on_sc=True, needs_layout_passes=False
    ),
)
def _gather_kernel(ints_t, tw, out_hbm, ibuf, gbuf, tr, gsem, wsem):
    w = lax.axis_index("s") * NC + lax.axis_index("c")
    lanes = lax.iota(jnp.int32, 16)

    for m in range(SUP_W):
        sup = w * SUP_W + m
        tb8 = (sup // NBB) * TB
        bb = sup % NBB
        pltpu.sync_copy(
            ints_t.at[pl.ds(tb8, TB), pl.ds(bb * G, G)],
            ibuf.at[pl.ds(m * TB, TB), :],
        )

    def issue_gather(s):
        pltpu.async_copy(tw.at[ibuf.at[s]], gbuf.at[lax.rem(s, 3)], gsem)

    issue_gather(0)
    issue_gather(1)

    def body(s, carry):
        bt = lax.rem(s, 2)
        sup = w * SUP_W + s // TB
        t = (sup // NBB) * TB + lax.rem(s, TB)
        bb = lax.rem(sup, NBB)

        @pl.when(s >= 2)
        def _():
            pltpu.make_async_copy(
                tr.at[bt], out_hbm.at[0, :, pl.ds(0, G)], wsem
            ).wait()

        @pl.when(s + 2 < STEPS)
        def _():
            issue_gather(s + 2)

        pltpu.make_async_copy(
            tw.at[ibuf.at[s]], gbuf.at[lax.rem(s, 3)], gsem
        ).wait()

        src = gbuf.at[lax.rem(s, 3)]
        dst = tr.at[bt]
        diags = [jnp.bitwise_and(lanes + k, 15) for k in range(16)]

        @pl.loop(0, G, step=16)
        def _(i0):
            for d0 in range(0, D, 16):
                cv = lanes + d0
                for k0 in range(0, 16, 8):
                    rvs = [diags[k0 + k] + i0 for k in range(8)]
                    vals = [
                        plsc.load_gather(src, [rvs[k], cv]) for k in range(8)
                    ]
                    for k in range(8):
                        plsc.store_scatter(dst, [cv, rvs[k]], vals[k])

        pltpu.async_copy(dst, out_hbm.at[t, :, pl.ds(bb * G, G)], wsem)
        return carry

    lax.fori_loop(0, STEPS, body, 0)

    pltpu.make_async_copy(tr.at[0], out_hbm.at[0, :, pl.ds(0, G)], wsem).wait()
    pltpu.make_async_copy(tr.at[0], out_hbm.at[0, :, pl.ds(0, G)], wsem).wait()


NBLK = (V + 127) // 128


@functools.partial(
    pl.kernel,
    out_type=jax.ShapeDtypeStruct((NBLK * 128, 128), jnp.float32),
    mesh=plsc.VectorSubcoreMesh(core_axis_name="c", subcore_axis_name="s"),
    scratch_types=[
        pltpu.VMEM((3, D, 128), jnp.float32),
        pltpu.VMEM((2, 128, 128), jnp.float32),
        pltpu.SemaphoreType.DMA,
        pltpu.SemaphoreType.DMA,
    ],
    compiler_params=pltpu.CompilerParams(
        use_tc_tiling_on_sc=True,
        needs_layout_passes=False,
        disable_bounds_checks=True,
    ),
)
def _pack_kernel(tt, tp, vin, vout, rsem, wsem):
    w = lax.axis_index("s") * NC + lax.axis_index("c")
    lanes = lax.iota(jnp.int32, 16)
    base, extra = divmod(NBLK, NW)
    cnt = base + jnp.where(w < extra, 1, 0)
    first = base * w + jnp.minimum(w, extra)

    def col0(i):
        return pl.multiple_of((first + i) * 128, 128)

    def issue_read(i):
        pltpu.async_copy(
            tt.at[:, pl.ds(col0(i), 128)], vin.at[lax.rem(i, 3)], rsem
        )

    issue_read(0)

    @pl.when(cnt > 1)
    def _():
        issue_read(1)

    def body(i, carry):
        b = lax.rem(i, 2)

        @pl.when(i + 2 < cnt)
        def _():
            issue_read(i + 2)

        pltpu.make_async_copy(
            tt.at[:, pl.ds(col0(i), 128)], vin.at[lax.rem(i, 3)], rsem
        ).wait()

        @pl.when(i >= 2)
        def _():
            pltpu.make_async_copy(vout.at[b], tp.at[pl.ds(0, 128), :], wsem).wait()

        src = vin.at[lax.rem(i, 3)]
        dstp = vout.at[b]
        diags = [jnp.bitwise_and(lanes + k, 15) for k in range(16)]

        @pl.loop(0, D, step=16)
        def _(x0):
            rvl = lanes + x0
            rvs = lanes + x0
            for j0 in range(0, 128, 16):
                for k0 in range(0, 16, 8):
                    cvs = [diags[k0 + k] + j0 for k in range(8)]
                    vals = [
                        plsc.load_gather(src, [rvl, cvs[k]]) for k in range(8)
                    ]
                    for k in range(8):
                        plsc.store_scatter(dstp, [cvs[k], rvs], vals[k])

        pltpu.async_copy(dstp, tp.at[pl.ds(col0(i), 128), :], wsem)
        return carry

    lax.fori_loop(0, cnt, body, 0)

    pltpu.make_async_copy(vout.at[0], tp.at[pl.ds(0, 128), :], wsem).wait()
    pltpu.make_async_copy(vout.at[0], tp.at[pl.ds(0, 128), :], wsem).wait()


def kernel(integers, table):
    ints_t = integers.T
    tt = jnp.swapaxes(table, 0, 1)
    tp = _pack_kernel(tt)
    out = _gather_kernel(ints_t, tp)
    return out.transpose(2, 0, 1)
